# baseline (speedup 1.0000x reference)
import functools

import jax
import jax.numpy as jnp
from jax import lax
from jax.experimental import pallas as pl
from jax.experimental.pallas import tpu as pltpu

BT = 128
LP = 128


def _make_body(H, W, Bt, Cin):
    HW = H * W
    HWP = HW + 2 * LP

    def body(x_ref, p_ref, ye_ref, ys_ref, ee_ref):
        x2 = x_ref[...].reshape(Bt * Cin, HW).astype(jnp.bfloat16)
        zpad = jnp.zeros((Bt * Cin, LP), jnp.bfloat16)
        xp = jnp.concatenate([zpad, x2, zpad], axis=1)

        pos = lax.broadcasted_iota(jnp.int32, (1, HWP), 1)
        w_idx = (pos - LP) % W
        zero = jnp.zeros((), jnp.bfloat16)
        zm1 = jnp.where(w_idx >= 1, pltpu.roll(xp, shift=1, axis=1), zero)
        zp1 = jnp.where(w_idx <= W - 2,
                        pltpu.roll(xp, shift=HWP - 1, axis=1), zero)
        mids = (zm1, xp, zp1)
        ups = tuple(pltpu.roll(z, shift=W, axis=1) for z in mids)
        downs = tuple(pltpu.roll(z, shift=HWP - W, axis=1) for z in mids)
        groups = ups + mids + downs

        wfull = p_ref[:, 0:73].astype(jnp.bfloat16)
        ones_row = jnp.ones((1, HW), jnp.bfloat16)
        inv_hw = 1.0 / float(HW)

        pooled_cols = []
        for b in range(Bt):
            rows = [g[b * Cin:(b + 1) * Cin, LP:LP + HW] for g in groups]
            patches = jnp.concatenate(rows + [ones_row], axis=0)
            rT = jnp.dot(wfull, patches,
                         preferred_element_type=jnp.float32)
            rT = jnp.maximum(rT, 0.0)
            pooled_cols.append(jnp.sum(rT, axis=1, keepdims=True))

        pooledT = jnp.concatenate(pooled_cols, axis=1) * inv_hw

        def sigmoid(z):
            return 0.5 * (jnp.tanh(0.5 * z) + 1.0)

        pooled_bf = pooledT.astype(jnp.bfloat16)
        eeT = sigmoid(jnp.dot(p_ref[:, 128:256].astype(jnp.bfloat16),
                              pooled_bf,
                              preferred_element_type=jnp.float32)
                      + p_ref[:, 512:513])
        logits = (jnp.dot(p_ref[:, 256:384].astype(jnp.bfloat16),
                          pooled_bf,
                          preferred_element_type=jnp.float32)
                  + p_ref[:, 513:514])
        ex = jnp.exp(logits - jnp.max(logits, axis=0, keepdims=True))
        ysT = ex / jnp.sum(ex, axis=0, keepdims=True)
        mseT = sigmoid(jnp.dot(p_ref[:, 384:512].astype(jnp.bfloat16),
                               ysT.astype(jnp.bfloat16),
                               preferred_element_type=jnp.float32))
        yeT = eeT * mseT

        ye_ref[...] = yeT.T[:, :ye_ref.shape[1]]
        ys_ref[...] = ysT.T[:, :ys_ref.shape[1]]
        ee_ref[...] = eeT.T[:, :ee_ref.shape[1]]

    return body


@functools.partial(jax.jit, static_argnames=("sed_class", "asc_class"))
def _forward(x, wconvT, bconv, wsed, bsed, wasc, basc, dse, *,
             sed_class, asc_class):
    B, Cin, H, W = x.shape
    HW = H * W
    x_flat = x
    steps = pl.cdiv(B, BT)
    B_pad = steps * BT
    if B_pad != B:
        x_flat = jnp.concatenate(
            [x_flat, jnp.zeros((B_pad - B, Cin, H, W), x_flat.dtype)],
            axis=0)

    cout = wconvT.shape[0]
    packed = jnp.concatenate([
        wconvT,
        bconv,
        jnp.zeros((cout, 55), jnp.float32),
        wsed.T, wasc.T, dse.T,
        bsed.T, basc.T,
        jnp.zeros((cout, 126), jnp.float32),
    ], axis=1)

    out_shapes = (jax.ShapeDtypeStruct((B_pad, sed_class), jnp.float32),
                  jax.ShapeDtypeStruct((B_pad, asc_class), jnp.float32),
                  jax.ShapeDtypeStruct((B_pad, sed_class), jnp.float32))
    out_specs = tuple(pl.BlockSpec((BT, n), lambda i: (i, 0))
                      for n in (sed_class, asc_class, sed_class))

    y_e, y_s, e_e = pl.pallas_call(
        _make_body(H, W, BT, Cin),
        out_shape=out_shapes,
        grid=(steps,),
        in_specs=[
            pl.BlockSpec((BT, Cin, H, W), lambda i: (i, 0, 0, 0)),
            pl.BlockSpec(packed.shape, lambda i: (0, 0)),
        ],
        out_specs=out_specs,
        compiler_params=pltpu.CompilerParams(
            dimension_semantics=("parallel",)),
    )(x_flat, packed)

    if B_pad != B:
        y_e, y_s, e_e = y_e[:B], y_s[:B], e_e[:B]
    return (y_e, y_s, e_e)


def kernel(x, wconvT, bconv, wsed, bsed, wasc, basc, dse):
    return _forward(x, wconvT, bconv, wsed, bsed, wasc, basc, dse,
                    sed_class=64, asc_class=32)


# speedup vs baseline: 1.7708x; 1.7708x over previous
import functools

import jax
import jax.numpy as jnp
from jax import lax
from jax.experimental import pallas as pl
from jax.experimental.pallas import tpu as pltpu

BT = 128
LP = 128


def _make_body(H, W, Bt, Cin):
    HW = H * W
    HWP = HW + 2 * LP

    def body(x_ref, p_ref, ye_ref, ys_ref, ee_ref):
        x2 = x_ref[...].reshape(Bt * Cin, HW).astype(jnp.bfloat16)
        zpad = jnp.zeros((Bt * Cin, LP), jnp.bfloat16)
        xp = jnp.concatenate([zpad, x2, zpad], axis=1)

        pos = lax.broadcasted_iota(jnp.int32, (1, HWP), 1)
        w_idx = (pos - LP) % W
        zero = jnp.zeros((), jnp.bfloat16)
        zm1 = jnp.where(w_idx >= 1, pltpu.roll(xp, shift=1, axis=1), zero)
        zp1 = jnp.where(w_idx <= W - 2,
                        pltpu.roll(xp, shift=HWP - 1, axis=1), zero)
        mids = (zm1, xp, zp1)
        ups = tuple(pltpu.roll(z, shift=W, axis=1) for z in mids)
        downs = tuple(pltpu.roll(z, shift=HWP - W, axis=1) for z in mids)
        groups = ups + mids + downs

        wfull = p_ref[:, 0:73].astype(jnp.bfloat16)
        ones_row = jnp.ones((1, HW), jnp.bfloat16)
        inv_hw = 1.0 / float(HW)

        pooled_cols = []
        for b in range(Bt):
            rows = [g[b * Cin:(b + 1) * Cin, LP:LP + HW] for g in groups]
            patches = jnp.concatenate(rows + [ones_row], axis=0)
            rT = jnp.dot(wfull, patches,
                         preferred_element_type=jnp.float32)
            rT = jnp.maximum(rT, 0.0)
            pooled_cols.append(jnp.sum(rT, axis=1, keepdims=True))

        pooledT = jnp.concatenate(pooled_cols, axis=1) * inv_hw

        def sigmoid(z):
            return 0.5 * (jnp.tanh(0.5 * z) + 1.0)

        pooled_bf = pooledT.astype(jnp.bfloat16)
        eeT = sigmoid(jnp.dot(p_ref[:, 128:256].astype(jnp.bfloat16),
                              pooled_bf,
                              preferred_element_type=jnp.float32)
                      + p_ref[:, 512:513])
        logits = (jnp.dot(p_ref[:, 256:384].astype(jnp.bfloat16),
                          pooled_bf,
                          preferred_element_type=jnp.float32)
                  + p_ref[:, 513:514])
        ex = jnp.exp(logits - jnp.max(logits, axis=0, keepdims=True))
        ysT = ex / jnp.sum(ex, axis=0, keepdims=True)
        mseT = sigmoid(jnp.dot(p_ref[:, 384:512].astype(jnp.bfloat16),
                               ysT.astype(jnp.bfloat16),
                               preferred_element_type=jnp.float32))
        yeT = eeT * mseT

        ye_ref[...] = yeT.T[:, :ye_ref.shape[1]]
        ys_ref[...] = ysT.T[:, :ys_ref.shape[1]]
        ee_ref[...] = eeT.T[:, :ee_ref.shape[1]]

    return body


@functools.partial(jax.jit, static_argnames=("sed_class", "asc_class"))
def _forward(x, wconvT, bconv, wsed, bsed, wasc, basc, dse, *,
             sed_class, asc_class):
    B, Cin, H, W = x.shape
    HW = H * W
    x_flat = x.reshape(B, Cin, HW).astype(jnp.float32)

    steps = pl.cdiv(B, BT)
    B_pad = steps * BT
    if B_pad != B:
        x_flat = jnp.concatenate(
            [x_flat, jnp.zeros((B_pad - B, Cin, HW), x_flat.dtype)], axis=0)

    cout = wconvT.shape[0]
    packed = jnp.concatenate([
        wconvT,
        bconv,
        jnp.zeros((cout, 55), jnp.float32),
        wsed.T, wasc.T, dse.T,
        bsed.T, basc.T,
        jnp.zeros((cout, 126), jnp.float32),
    ], axis=1)

    out_shapes = (jax.ShapeDtypeStruct((B_pad, sed_class), jnp.float32),
                  jax.ShapeDtypeStruct((B_pad, asc_class), jnp.float32),
                  jax.ShapeDtypeStruct((B_pad, sed_class), jnp.float32))
    out_specs = tuple(pl.BlockSpec((BT, n), lambda i: (i, 0))
                      for n in (sed_class, asc_class, sed_class))

    y_e, y_s, e_e = pl.pallas_call(
        _make_body(H, W, BT, Cin),
        out_shape=out_shapes,
        grid=(steps,),
        in_specs=[
            pl.BlockSpec((BT, Cin, HW), lambda i: (i, 0, 0)),
            pl.BlockSpec(packed.shape, lambda i: (0, 0)),
        ],
        out_specs=out_specs,
        compiler_params=pltpu.CompilerParams(
            dimension_semantics=("parallel",)),
    )(x_flat, packed)

    if B_pad != B:
        y_e, y_s, e_e = y_e[:B], y_s[:B], e_e[:B]
    return (y_e, y_s, e_e)


def kernel(x, wconvT, bconv, wsed, bsed, wasc, basc, dse):
    return _forward(x, wconvT, bconv, wsed, bsed, wasc, basc, dse,
                    sed_class=64, asc_class=32)


# speedup vs baseline: 1.8597x; 1.0502x over previous
import functools

import jax
import jax.numpy as jnp
from jax import lax
from jax.experimental import pallas as pl
from jax.experimental.pallas import tpu as pltpu

BT = 128
LP = 128


def _make_body(H, W, Bt, Cin):
    HW = H * W
    HWP = HW + 2 * LP

    def body(x_ref, p_ref, ye_ref, ys_ref, ee_ref):
        x2 = x_ref[...].reshape(Bt * Cin, HW).astype(jnp.bfloat16)
        zpad = jnp.zeros((Bt * Cin, LP), jnp.bfloat16)
        xp = jnp.concatenate([zpad, x2, zpad], axis=1)

        pos = lax.broadcasted_iota(jnp.int32, (1, HWP), 1)
        w_idx = (pos - LP) % W
        mm1 = (w_idx >= 1).astype(jnp.bfloat16)
        mp1 = (w_idx <= W - 2).astype(jnp.bfloat16)
        zm1 = pltpu.roll(xp, shift=1, axis=1) * mm1
        zp1 = pltpu.roll(xp, shift=HWP - 1, axis=1) * mp1
        mids = (zm1, xp, zp1)
        ups = tuple(pltpu.roll(z, shift=W, axis=1) for z in mids)
        downs = tuple(pltpu.roll(z, shift=HWP - W, axis=1) for z in mids)
        groups = ups + mids + downs

        wfull = p_ref[:, 0:73].astype(jnp.bfloat16)
        ones_row = jnp.ones((1, HW), jnp.bfloat16)
        inv_hw = 1.0 / float(HW)

        pooled_cols = []
        for b in range(Bt):
            rows = [g[b * Cin:(b + 1) * Cin, LP:LP + HW] for g in groups]
            patches = jnp.concatenate(rows + [ones_row], axis=0)
            rT = jnp.dot(wfull, patches,
                         preferred_element_type=jnp.float32)
            rT = jnp.maximum(rT, 0.0)
            pooled_cols.append(jnp.sum(rT, axis=1, keepdims=True))

        pooledT = jnp.concatenate(pooled_cols, axis=1) * inv_hw

        def sigmoid(z):
            return 0.5 * (jnp.tanh(0.5 * z) + 1.0)

        pooled_bf = pooledT.astype(jnp.bfloat16)
        eeT = sigmoid(jnp.dot(p_ref[:, 128:256].astype(jnp.bfloat16),
                              pooled_bf,
                              preferred_element_type=jnp.float32)
                      + p_ref[:, 512:513])
        logits = (jnp.dot(p_ref[:, 256:384].astype(jnp.bfloat16),
                          pooled_bf,
                          preferred_element_type=jnp.float32)
                  + p_ref[:, 513:514])
        ex = jnp.exp(logits - jnp.max(logits, axis=0, keepdims=True))
        ysT = ex / jnp.sum(ex, axis=0, keepdims=True)
        mseT = sigmoid(jnp.dot(p_ref[:, 384:512].astype(jnp.bfloat16),
                               ysT.astype(jnp.bfloat16),
                               preferred_element_type=jnp.float32))
        yeT = eeT * mseT

        ye_ref[...] = yeT.T[:, :ye_ref.shape[1]]
        ys_ref[...] = ysT.T[:, :ys_ref.shape[1]]
        ee_ref[...] = eeT.T[:, :ee_ref.shape[1]]

    return body


@functools.partial(jax.jit, static_argnames=("sed_class", "asc_class"))
def _forward(x, wconvT, bconv, wsed, bsed, wasc, basc, dse, *,
             sed_class, asc_class):
    B, Cin, H, W = x.shape
    HW = H * W
    x_flat = x.reshape(B, Cin, HW).astype(jnp.float32)

    steps = pl.cdiv(B, BT)
    B_pad = steps * BT
    if B_pad != B:
        x_flat = jnp.concatenate(
            [x_flat, jnp.zeros((B_pad - B, Cin, HW), x_flat.dtype)], axis=0)

    cout = wconvT.shape[0]
    packed = jnp.concatenate([
        wconvT,
        bconv,
        jnp.zeros((cout, 55), jnp.float32),
        wsed.T, wasc.T, dse.T,
        bsed.T, basc.T,
        jnp.zeros((cout, 126), jnp.float32),
    ], axis=1)

    out_shapes = (jax.ShapeDtypeStruct((B_pad, sed_class), jnp.float32),
                  jax.ShapeDtypeStruct((B_pad, asc_class), jnp.float32),
                  jax.ShapeDtypeStruct((B_pad, sed_class), jnp.float32))
    out_specs = tuple(pl.BlockSpec((BT, n), lambda i: (i, 0))
                      for n in (sed_class, asc_class, sed_class))

    y_e, y_s, e_e = pl.pallas_call(
        _make_body(H, W, BT, Cin),
        out_shape=out_shapes,
        grid=(steps,),
        in_specs=[
            pl.BlockSpec((BT, Cin, HW), lambda i: (i, 0, 0)),
            pl.BlockSpec(packed.shape, lambda i: (0, 0)),
        ],
        out_specs=out_specs,
        compiler_params=pltpu.CompilerParams(
            dimension_semantics=("parallel",)),
    )(x_flat, packed)

    if B_pad != B:
        y_e, y_s, e_e = y_e[:B], y_s[:B], e_e[:B]
    return (y_e, y_s, e_e)


def kernel(x, wconvT, bconv, wsed, bsed, wasc, basc, dse):
    return _forward(x, wconvT, bconv, wsed, bsed, wasc, basc, dse,
                    sed_class=64, asc_class=32)
